# baseline (speedup 1.0000x reference)
import jax
import jax.numpy as jnp
from jax.experimental import pallas as pl

_BLOCK_ROWS = 5000


def _fused_cell_body(x_ref, wz_ref, bz_ref, wh_ref, bh_ref, wl_ref, bl_ref,
                     o_ref):
    f_in = x_ref.shape[1]
    az = 0.5 * (wz_ref[0, :f_in, :] + wz_ref[1, :f_in, :])
    ah = wh_ref[0, :f_in, :] + wh_ref[1, :f_in, :]
    aw = jnp.concatenate([az, ah], axis=1)
    f_out = az.shape[1]

    xb = x_ref[...].astype(jnp.bfloat16)
    r = jnp.dot(xb, aw.astype(jnp.bfloat16),
                preferred_element_type=jnp.float32)
    t1 = jnp.tanh(r[:, :f_out] + 0.5 * bz_ref[...])
    t2 = jnp.tanh(r[:, f_out:] + bh_ref[...])
    h = jnp.maximum((1.0 - t1) * t2, 0.0)
    o_ref[...] = (
        jnp.dot(h.astype(jnp.bfloat16),
                (0.5 * wl_ref[...]).astype(jnp.bfloat16),
                preferred_element_type=jnp.float32)
        + bl_ref[...])


def kernel(x, edge_index, edge_weight, Wz, bz, Wr, br, Wh, bh, Wl, bl):
    del edge_index, edge_weight, Wr, br
    n, f_in = x.shape
    c_in = Wz.shape[-2]
    f_out = Wz.shape[-1]

    wz = Wz.reshape(2, c_in, f_out)
    wh = Wh.reshape(2, c_in, f_out)
    bz2 = bz.reshape(1, f_out)
    bh2 = bh.reshape(1, f_out)
    wl = Wl.reshape(f_out, 1)
    bl2 = bl.reshape(1, 1)

    grid = (pl.cdiv(n, _BLOCK_ROWS),)
    return pl.pallas_call(
        _fused_cell_body,
        grid=grid,
        in_specs=[
            pl.BlockSpec((_BLOCK_ROWS, f_in), lambda i: (i, 0)),
            pl.BlockSpec((2, c_in, f_out), lambda i: (0, 0, 0)),
            pl.BlockSpec((1, f_out), lambda i: (0, 0)),
            pl.BlockSpec((2, c_in, f_out), lambda i: (0, 0, 0)),
            pl.BlockSpec((1, f_out), lambda i: (0, 0)),
            pl.BlockSpec((f_out, 1), lambda i: (0, 0)),
            pl.BlockSpec((1, 1), lambda i: (0, 0)),
        ],
        out_specs=pl.BlockSpec((_BLOCK_ROWS, 1), lambda i: (i, 0)),
        out_shape=jax.ShapeDtypeStruct((n, 1), x.dtype),
    )(x, wz, bz2, wh, bh2, wl, bl2)


# speedup vs baseline: 1.0023x; 1.0023x over previous
import jax
import jax.numpy as jnp
from jax.experimental import pallas as pl

_BLOCK_ROWS = 5000


def _fused_cell_body(x_ref, wz_ref, bz_ref, wh_ref, bh_ref, wl_ref, bl_ref,
                     o_ref):
    f_in = x_ref.shape[1]
    az = 0.5 * (wz_ref[0, :f_in, :] + wz_ref[1, :f_in, :])
    ah = wh_ref[0, :f_in, :] + wh_ref[1, :f_in, :]
    aw = jnp.concatenate([az, ah], axis=1)
    f_out = az.shape[1]

    bf = jnp.bfloat16
    xb = x_ref[...].astype(bf)
    r = jnp.dot(xb, aw.astype(bf),
                preferred_element_type=jnp.float32).astype(bf)
    t1 = jnp.tanh(r[:, :f_out] + (0.5 * bz_ref[...]).astype(bf))
    t2 = jnp.tanh(r[:, f_out:] + bh_ref[...].astype(bf))
    h = jnp.maximum((1.0 - t1) * t2, jnp.array(0.0, bf))
    o_ref[...] = (
        jnp.dot(h, (0.5 * wl_ref[...]).astype(bf),
                preferred_element_type=jnp.float32)
        + bl_ref[...])


def kernel(x, edge_index, edge_weight, Wz, bz, Wr, br, Wh, bh, Wl, bl):
    del edge_index, edge_weight, Wr, br
    n, f_in = x.shape
    c_in = Wz.shape[-2]
    f_out = Wz.shape[-1]

    wz = Wz.reshape(2, c_in, f_out)
    wh = Wh.reshape(2, c_in, f_out)
    bz2 = bz.reshape(1, f_out)
    bh2 = bh.reshape(1, f_out)
    wl = Wl.reshape(f_out, 1)
    bl2 = bl.reshape(1, 1)

    grid = (pl.cdiv(n, _BLOCK_ROWS),)
    return pl.pallas_call(
        _fused_cell_body,
        grid=grid,
        in_specs=[
            pl.BlockSpec((_BLOCK_ROWS, f_in), lambda i: (i, 0)),
            pl.BlockSpec((2, c_in, f_out), lambda i: (0, 0, 0)),
            pl.BlockSpec((1, f_out), lambda i: (0, 0)),
            pl.BlockSpec((2, c_in, f_out), lambda i: (0, 0, 0)),
            pl.BlockSpec((1, f_out), lambda i: (0, 0)),
            pl.BlockSpec((f_out, 1), lambda i: (0, 0)),
            pl.BlockSpec((1, 1), lambda i: (0, 0)),
        ],
        out_specs=pl.BlockSpec((_BLOCK_ROWS, 1), lambda i: (i, 0)),
        out_shape=jax.ShapeDtypeStruct((n, 1), x.dtype),
    )(x, wz, bz2, wh, bh2, wl, bl2)
